# baseline (speedup 1.0000x reference)
import jax
import jax.numpy as jnp
from jax.experimental import pallas as pl
from jax.experimental.pallas import tpu as pltpu

MODAL_NODES = 2
STEP_DOMAIN = 0

_BLOCK_R = 256
_NBUF = 8


def _adj_stream_kernel(o_ref, scratch_ref, sem_ref, *, b, n):
    nfull = b // _BLOCK_R
    cols = jax.lax.broadcasted_iota(jnp.int32, (_BLOCK_R, n), 1)

    def copy_for(k):
        slot = k % _NBUF
        return pltpu.make_async_copy(
            scratch_ref.at[slot],
            o_ref.at[pl.ds(k * _BLOCK_R, _BLOCK_R)],
            sem_ref.at[slot],
        )

    for k in range(nfull):
        slot = k % _NBUF
        if k >= _NBUF:
            copy_for(k - _NBUF).wait()
        rows = (
            jax.lax.broadcasted_iota(jnp.int32, (_BLOCK_R, n), 0)
            + k * _BLOCK_R
        )
        scratch_ref[slot] = (rows == cols).astype(jnp.float32)
        copy_for(k).start()

    for k in range(max(nfull - _NBUF, 0), nfull):
        copy_for(k).wait()

    tail = n - b
    scratch_ref[0, :8] = (cols[:8] >= b).astype(jnp.float32)
    tail_cp = pltpu.make_async_copy(
        scratch_ref.at[0, :tail],
        o_ref.at[pl.ds(b, tail)],
        sem_ref.at[0],
    )
    tail_cp.start()
    tail_cp.wait()


def kernel(x, step):
    del step
    B, _ = x.shape
    N = B + MODAL_NODES * (STEP_DOMAIN + 1) + 1 + STEP_DOMAIN
    import functools
    body = functools.partial(_adj_stream_kernel, b=B, n=N)
    return pl.pallas_call(
        body,
        out_specs=pl.BlockSpec(memory_space=pltpu.MemorySpace.HBM),
        out_shape=jax.ShapeDtypeStruct((N, N), jnp.float32),
        scratch_shapes=[
            pltpu.VMEM((_NBUF, _BLOCK_R, N), jnp.float32),
            pltpu.SemaphoreType.DMA((_NBUF,)),
        ],
    )()


# speedup vs baseline: 1.0954x; 1.0954x over previous
import jax
import jax.numpy as jnp
from jax.experimental import pallas as pl

MODAL_NODES = 2
STEP_DOMAIN = 0

_BLOCK_R = 256


def _adj_block_kernel(o_ref, *, block_r, b):
    i = pl.program_id(0)
    r0 = i * block_r
    rows = jax.lax.broadcasted_iota(jnp.int32, o_ref.shape, 0) + r0
    cols = jax.lax.broadcasted_iota(jnp.int32, o_ref.shape, 1)
    hit = (rows == cols) | ((rows >= b) & (cols >= b))
    o_ref[...] = hit.astype(jnp.float32)


def kernel(x, step):
    del step
    B, _ = x.shape
    N = B + MODAL_NODES * (STEP_DOMAIN + 1) + 1 + STEP_DOMAIN
    grid = (pl.cdiv(N, _BLOCK_R),)
    import functools
    body = functools.partial(_adj_block_kernel, block_r=_BLOCK_R, b=B)
    return pl.pallas_call(
        body,
        grid=grid,
        out_specs=pl.BlockSpec((_BLOCK_R, N), lambda i: (i, 0)),
        out_shape=jax.ShapeDtypeStruct((N, N), jnp.float32),
    )()
